# Initial kernel scaffold; baseline (speedup 1.0000x reference)
import jax
import jax.numpy as jnp
from jax.experimental import pallas as pl


def kernel(use_small_u, small_scales, large_scales, start_u):
    raise NotImplementedError("write your pallas kernel here")


# speedup vs baseline: 1.0148x; 1.0148x over previous
import functools

import jax
import jax.numpy as jnp
from jax import lax
from jax.experimental import pallas as pl
from jax.experimental.pallas import tpu as pltpu
from jax.experimental.pallas import tpu_sc as plsc

_SEQ = 4096
_ROWS = 16
_SPANS = 4
_NW = 32
_CHUNK = (_ROWS * _SEQ) // _NW
_STEPS = _CHUNK // 16


def _lane_scalar(v, lane, k):
    return jnp.sum(jnp.where(lane == k, v, jnp.int32(0)))


def _body(u_hbm, ss_hbm, ls_hbm, su_hbm, pos_hbm, tm_hbm, cm_hbm,
          u_v, ss_v, ls_v, su_v, pos_v, tm_v, cm_v):
    wid = lax.axis_index("s") * 2 + lax.axis_index("c")
    row = wid // 2
    half = wid % 2
    base = half * _CHUNK

    pltpu.sync_copy(u_hbm, u_v)
    pltpu.sync_copy(ss_hbm, ss_v)
    pltpu.sync_copy(ls_hbm, ls_v)
    pltpu.sync_copy(su_hbm, su_v)

    lane = lax.broadcasted_iota(jnp.int32, (16,), 0)
    chunk = (row // 4) * 16
    l0 = (row % 4) * _SPANS
    u = u_v[pl.ds(chunk, 16)]
    ss = ss_v[pl.ds(chunk, 16)]
    ls = ls_v[pl.ds(chunk, 16)]
    su = su_v[pl.ds(chunk, 16)]

    scales = jnp.where(u < 0.5, ss, ls)
    span_lens = jnp.maximum((scales * float(_SEQ)).astype(jnp.int32), 1)
    max_starts = jnp.maximum(_SEQ - span_lens, 0)
    starts = (su * (max_starts.astype(jnp.float32) + 1.0)).astype(jnp.int32)
    ends = jnp.minimum(starts + span_lens, _SEQ)

    valid = (lane >= l0) & (lane < l0 + _SPANS)
    key = jnp.where(valid, starts, jnp.int32(2 ** 30))
    val = jnp.where(valid, ends, jnp.int32(0))
    s_sorted, e_sorted = plsc.sort_key_val(key, val)

    run_end = jnp.int32(0)
    cum = jnp.int32(0)
    spans = []
    for k in range(_SPANS):
        sk = _lane_scalar(s_sorted, lane, k)
        ek = _lane_scalar(e_sorted, lane, k)
        s_clip = jnp.maximum(sk, run_end)
        ln = jnp.maximum(ek - s_clip, 0)
        spans.append((s_clip, s_clip + ln, cum, cum + ln, s_clip - cum))
        run_end = jnp.maximum(run_end, ek)
        cum = cum + ln

    one = jnp.int32(1)
    zero = jnp.int32(0)

    def step(i, carry):
        j = base + i * 16 + lane
        t = (j >= spans[0][0]) & (j < spans[0][1])
        p = jnp.full((16,), _SEQ, jnp.int32)
        for ms, me, clo, chi, off in spans:
            t = t | ((j >= ms) & (j < me))
            p = jnp.where((j >= clo) & (j < chi), j + off, p)
        tm_v[pl.ds(i * 16, 16)] = jnp.where(t, one, zero)
        cm_v[pl.ds(i * 16, 16)] = jnp.where(t, zero, one)
        pos_v[pl.ds(i * 16, 16)] = p
        return carry

    lax.fori_loop(0, _STEPS, step, 0, unroll=4)

    pltpu.sync_copy(pos_v, pos_hbm.at[row, pl.ds(base, _CHUNK)])
    pltpu.sync_copy(tm_v, tm_hbm.at[row, pl.ds(base, _CHUNK)])
    pltpu.sync_copy(cm_v, cm_hbm.at[row, pl.ds(base, _CHUNK)])


_span_masks = functools.partial(
    pl.kernel,
    out_type=(
        jax.ShapeDtypeStruct((_ROWS, _SEQ), jnp.int32),
        jax.ShapeDtypeStruct((_ROWS, _SEQ), jnp.int32),
        jax.ShapeDtypeStruct((_ROWS, _SEQ), jnp.int32),
    ),
    mesh=plsc.VectorSubcoreMesh(core_axis_name="c", subcore_axis_name="s"),
    compiler_params=pltpu.CompilerParams(needs_layout_passes=False),
    scratch_types=[
        pltpu.VMEM((_ROWS * _SPANS,), jnp.float32),
        pltpu.VMEM((_ROWS * _SPANS,), jnp.float32),
        pltpu.VMEM((_ROWS * _SPANS,), jnp.float32),
        pltpu.VMEM((_ROWS * _SPANS,), jnp.float32),
        pltpu.VMEM((_CHUNK,), jnp.int32),
        pltpu.VMEM((_CHUNK,), jnp.int32),
        pltpu.VMEM((_CHUNK,), jnp.int32),
    ],
)(_body)


@jax.jit
def kernel(use_small_u, small_scales, large_scales, start_u):
    pos, tm, cm = _span_masks(use_small_u, small_scales, large_scales, start_u)
    return (cm.astype(jnp.bool_), tm.astype(jnp.bool_), pos)
